# baseline (speedup 1.0000x reference)
import functools

import jax
import jax.numpy as jnp
from jax import lax
from jax.experimental import pallas as pl
from jax.experimental.pallas import tpu as pltpu
from jax.experimental.pallas import tpu_sc as plsc

_L = 16
_NC = 2
_NS = 16
_C = 80
_AW = 144


def _prep_body(feat_ref, g_ref, b_ref, wq_ref, bq_ref, wk_ref, wv_ref,
               q_ref, k_ref, v_ref):
    f = feat_ref[...]
    mu = jnp.mean(f, axis=0, keepdims=True)
    var = jnp.mean((f - mu) ** 2, axis=0, keepdims=True)
    x = (f - mu) * lax.rsqrt(var + 1e-5) * g_ref[...] + b_ref[...]
    dn = (((1,), (1,)), ((), ()))
    q_ref[...] = lax.dot_general(x, wq_ref[...], dn,
                                 precision=lax.Precision.HIGHEST) + bq_ref[...]
    k_ref[...] = lax.dot_general(x, wk_ref[...], dn,
                                 precision=lax.Precision.HIGHEST)
    v_ref[...] = lax.dot_general(x, wv_ref[...], dn,
                                 precision=lax.Precision.HIGHEST)


def _final_body(p_ref, o_ref):
    t = p_ref[0] + p_ref[1]
    s = t[:, 128:129]
    o_ref[...] = t[:, :128] / jnp.where(s > 0.0, s, 1.0)


_BLK = 25


def _make_sc_edge(N, E, D):
    per_tile = E // (_NC * _NS)
    chunks = per_tile // _C
    row_chunks = N // _C
    rr_iters = (row_chunks + _NS - 1) // _NS
    erows_per_tile = per_tile // _C
    dst_row0 = E // _C

    mesh = plsc.VectorSubcoreMesh(core_axis_name="c", subcore_axis_name="s",
                                  num_cores=_NC, num_subcores=_NS)

    @functools.partial(
        pl.kernel,
        out_type=jax.ShapeDtypeStruct((_NC, N, _AW), jnp.float32),
        mesh=mesh,
        scratch_types=[
            pltpu.VMEM((_BLK, _C), jnp.int32),
            pltpu.VMEM((_BLK, _C), jnp.int32),
            pltpu.VMEM((_C, D), jnp.float32),
            pltpu.VMEM((_C, D), jnp.float32),
            pltpu.VMEM((_C, _AW), jnp.float32),
            pltpu.VMEM((D,), jnp.float32),
            pltpu.VMEM((_C, _L), jnp.float32),
            pltpu.VMEM_SHARED((N, _AW), jnp.float32),
            pltpu.SemaphoreType.DMA,
            pltpu.SemaphoreType.DMA,
            pltpu.SemaphoreType.DMA,
        ],
        compiler_params=pltpu.CompilerParams(use_tc_tiling_on_sc=False,
                                             needs_layout_passes=False,
                                             disable_bounds_checks=True),
    )
    def sc_edge(q_hbm, k_hbm, v_hbm, edge_hbm, we_hbm, out_hbm,
                src_blk, dst_blk, qb, kb, ob, web, wbuf, acc,
                sem_i, sem_q, sem_kv):
        cid = lax.axis_index("c")
        sid = lax.axis_index("s")
        lane = lax.iota(jnp.int32, _L)
        zero16 = jnp.zeros((_L,), jnp.float32)
        rows_l = [p * _L + lane for p in range(_C // _L)]

        def zrow(r, _):
            for cc in range(_AW // _L):
                ob[r, pl.ds(cc * _L, _L)] = zero16
            return 0
        lax.fori_loop(0, _C, zrow, 0)
        for jj in range(rr_iters):
            c = jj * _NS + sid
            @pl.when(c < row_chunks)
            def _():
                pltpu.sync_copy(ob, acc.at[pl.ds(c * _C, _C)])

        pltpu.sync_copy(we_hbm.at[0], web)
        plsc.subcore_barrier()
        wevs = [web[pl.ds(j * _L, _L)] for j in range(D // _L)]
        lane0 = lane == 0

        erow0 = (cid * _NS + sid) * erows_per_tile

        def load_block(b):
            c1 = pltpu.async_copy(
                edge_hbm.at[pl.ds(erow0 + b * _BLK, _BLK)], src_blk, sem_i)
            c2 = pltpu.async_copy(
                edge_hbm.at[pl.ds(dst_row0 + erow0 + b * _BLK, _BLK)],
                dst_blk, sem_i)
            c1.wait()
            c2.wait()

        load_block(0)
        pltpu.async_copy(q_hbm.at[src_blk.at[0]], qb, sem_q)
        pltpu.async_copy(k_hbm.at[dst_blk.at[0]], kb, sem_kv)

        def chunk(i, _):
            b = i // _BLK
            jj = i - b * _BLK

            @pl.when(jnp.logical_and(jj == 0, i > 0))
            def _():
                load_block(b)
                pltpu.async_copy(q_hbm.at[src_blk.at[0]], qb, sem_q)
                pltpu.async_copy(k_hbm.at[dst_blk.at[0]], kb, sem_kv)

            pltpu.make_async_copy(q_hbm.at[src_blk.at[0]], qb, sem_q).wait()
            pltpu.make_async_copy(k_hbm.at[dst_blk.at[0]], kb, sem_kv).wait()

            def astep(t, _):
                for u in range(4):
                    e = t * 4 + u
                    acc = zero16
                    for j in range(D // _L):
                        qv = qb[e, pl.ds(j * _L, _L)]
                        kv = kb[e, pl.ds(j * _L, _L)]
                        sg = 1.0 / (1.0 + jnp.exp(-(qv + kv)))
                        acc = acc + wevs[j] * sg
                    s = jnp.sum(acc)
                    wv = jnp.exp(jnp.broadcast_to(s, (_L,)))
                    wbuf[e, :] = wv
                    ob[e, pl.ds(D, _L)] = jnp.where(lane0, wv, 0.0)
                return 0
            lax.fori_loop(0, _C // 4, astep, 0)

            cp_v = pltpu.async_copy(v_hbm.at[src_blk.at[jj]], kb, sem_kv)
            @pl.when((i + 1) % _BLK != 0)
            def _():
                pltpu.async_copy(q_hbm.at[src_blk.at[jj + 1]], qb, sem_q)
            cp_v.wait()

            def bstep(t, _):
                for u in range(4):
                    e = t * 4 + u
                    wv = wbuf[e, :]
                    for j in range(D // _L):
                        vv = kb[e, pl.ds(j * _L, _L)]
                        ob[e, pl.ds(j * _L, _L)] = vv * wv
                return 0
            lax.fori_loop(0, _C // 4, bstep, 0)

            pltpu.sync_copy(ob, acc.at[dst_blk.at[jj]], add=True)
            @pl.when((i + 1) % _BLK != 0)
            def _():
                pltpu.async_copy(k_hbm.at[dst_blk.at[jj + 1]], kb, sem_kv)
            return 0
        lax.fori_loop(0, chunks, chunk, 0)

        plsc.subcore_barrier()
        for jj in range(rr_iters):
            c = jj * _NS + sid
            @pl.when(c < row_chunks)
            def _():
                sl = pl.ds(c * _C, _C)
                pltpu.sync_copy(acc.at[sl], out_hbm.at[cid, sl])

    return sc_edge


def kernel(feat, edge_index, bn_gamma, bn_beta, Wq, bq, Wk, Wv, We):
    N, D = feat.shape
    E = edge_index.shape[1]
    q, k, v = pl.pallas_call(
        _prep_body,
        out_shape=(
            jax.ShapeDtypeStruct((N, D), jnp.float32),
            jax.ShapeDtypeStruct((N, D), jnp.float32),
            jax.ShapeDtypeStruct((N, D), jnp.float32),
        ),
    )(feat, bn_gamma.reshape(1, -1), bn_beta.reshape(1, -1),
      Wq, bq.reshape(1, -1), Wk, Wv)

    partials = _make_sc_edge(N, E, D)(q, k, v,
                                      edge_index.reshape(-1, _C), We)

    rst = pl.pallas_call(
        _final_body,
        out_shape=jax.ShapeDtypeStruct((N, D), jnp.float32),
    )(partials)
    return rst


# speedup vs baseline: 1.3368x; 1.3368x over previous
import functools

import jax
import jax.numpy as jnp
from jax import lax
from jax.experimental import pallas as pl
from jax.experimental.pallas import tpu as pltpu
from jax.experimental.pallas import tpu_sc as plsc

_L = 16
_NC = 2
_NS = 16
_C = 80
_AW = 144


def _prep_body(feat_ref, g_ref, b_ref, wq_ref, bq_ref, wk_ref, wv_ref,
               q_ref, k_ref, v_ref):
    f = feat_ref[...]
    mu = jnp.mean(f, axis=0, keepdims=True)
    var = jnp.mean((f - mu) ** 2, axis=0, keepdims=True)
    x = (f - mu) * lax.rsqrt(var + 1e-5) * g_ref[...] + b_ref[...]
    dn = (((1,), (1,)), ((), ()))
    q_ref[...] = lax.dot_general(x, wq_ref[...], dn,
                                 precision=lax.Precision.HIGHEST) + bq_ref[...]
    k_ref[...] = lax.dot_general(x, wk_ref[...], dn,
                                 precision=lax.Precision.HIGHEST)
    v_ref[...] = lax.dot_general(x, wv_ref[...], dn,
                                 precision=lax.Precision.HIGHEST)


def _final_body(p_ref, o_ref):
    t = p_ref[0] + p_ref[1]
    s = t[:, 128:129]
    o_ref[...] = t[:, :128] / jnp.where(s > 0.0, s, 1.0)


_BLK = 25


def _make_sc_edge(N, E, D):
    per_tile = E // (_NC * _NS)
    chunks = per_tile // _C
    row_chunks = N // _C
    rr_iters = (row_chunks + _NS - 1) // _NS
    erows_per_tile = per_tile // _C
    dst_row0 = E // _C

    mesh = plsc.VectorSubcoreMesh(core_axis_name="c", subcore_axis_name="s",
                                  num_cores=_NC, num_subcores=_NS)

    @functools.partial(
        pl.kernel,
        out_type=jax.ShapeDtypeStruct((_NC, N, _AW), jnp.float32),
        mesh=mesh,
        scratch_types=[
            pltpu.VMEM((_BLK, _C), jnp.int32),
            pltpu.VMEM((_BLK, _C), jnp.int32),
            pltpu.VMEM((_C, D), jnp.float32),
            pltpu.VMEM((_C, D), jnp.float32),
            pltpu.VMEM((_C, _AW), jnp.float32),
            pltpu.VMEM((D,), jnp.float32),
            pltpu.VMEM((_C, _L), jnp.float32),
            pltpu.VMEM_SHARED((N, _AW), jnp.float32),
            pltpu.SemaphoreType.DMA,
            pltpu.SemaphoreType.DMA,
            pltpu.SemaphoreType.DMA,
        ],
        compiler_params=pltpu.CompilerParams(use_tc_tiling_on_sc=False,
                                             needs_layout_passes=False,
                                             disable_bounds_checks=True),
    )
    def sc_edge(q_hbm, k_hbm, v_hbm, edge_hbm, we_hbm, out_hbm,
                src_blk, dst_blk, qb, kb, ob, web, wbuf, acc,
                sem_i, sem_q, sem_kv):
        cid = lax.axis_index("c")
        sid = lax.axis_index("s")
        lane = lax.iota(jnp.int32, _L)
        zero16 = jnp.zeros((_L,), jnp.float32)
        rows_l = [p * _L + lane for p in range(_C // _L)]

        def zrow(r, _):
            for cc in range(_AW // _L):
                ob[r, pl.ds(cc * _L, _L)] = zero16
            return 0
        lax.fori_loop(0, _C, zrow, 0)
        for jj in range(rr_iters):
            c = jj * _NS + sid
            @pl.when(c < row_chunks)
            def _():
                pltpu.sync_copy(ob, acc.at[pl.ds(c * _C, _C)])

        pltpu.sync_copy(we_hbm.at[0], web)
        plsc.subcore_barrier()
        wevs = [web[pl.ds(j * _L, _L)] for j in range(D // _L)]
        lane0 = lane == 0

        erow0 = (cid * _NS + sid) * erows_per_tile

        def load_block(b):
            c1 = pltpu.async_copy(
                edge_hbm.at[pl.ds(erow0 + b * _BLK, _BLK)], src_blk, sem_i)
            c2 = pltpu.async_copy(
                edge_hbm.at[pl.ds(dst_row0 + erow0 + b * _BLK, _BLK)],
                dst_blk, sem_i)
            c1.wait()
            c2.wait()

        load_block(0)
        pltpu.async_copy(q_hbm.at[src_blk.at[0]], qb, sem_q)
        pltpu.async_copy(k_hbm.at[dst_blk.at[0]], kb, sem_kv)

        def chunk(i, _):
            b = i // _BLK
            jj = i - b * _BLK

            @pl.when(jnp.logical_and(jj == 0, i > 0))
            def _():
                load_block(b)
                pltpu.async_copy(q_hbm.at[src_blk.at[0]], qb, sem_q)
                pltpu.async_copy(k_hbm.at[dst_blk.at[0]], kb, sem_kv)

            pltpu.make_async_copy(q_hbm.at[src_blk.at[0]], qb, sem_q).wait()
            pltpu.make_async_copy(k_hbm.at[dst_blk.at[0]], kb, sem_kv).wait()

            def astep(t, _):
                for u in range(2):
                    e = t * 2 + u
                    acc = zero16
                    for j in range(D // _L):
                        qv = qb[e, pl.ds(j * _L, _L)]
                        kv = kb[e, pl.ds(j * _L, _L)]
                        sg = 1.0 / (1.0 + jnp.exp(-(qv + kv)))
                        acc = acc + wevs[j] * sg
                    s = jnp.sum(acc)
                    wv = jnp.exp(jnp.broadcast_to(s, (_L,)))
                    wbuf[e, :] = wv
                    ob[e, pl.ds(D, _L)] = jnp.where(lane0, wv, 0.0)
                return 0
            lax.fori_loop(0, _C // 2, astep, 0)

            cp_v = pltpu.async_copy(v_hbm.at[src_blk.at[jj]], kb, sem_kv)
            @pl.when((i + 1) % _BLK != 0)
            def _():
                pltpu.async_copy(q_hbm.at[src_blk.at[jj + 1]], qb, sem_q)
            cp_v.wait()

            def bstep(t, _):
                for u in range(4):
                    e = t * 4 + u
                    wv = wbuf[e, :]
                    for j in range(D // _L):
                        vv = kb[e, pl.ds(j * _L, _L)]
                        ob[e, pl.ds(j * _L, _L)] = vv * wv
                return 0
            lax.fori_loop(0, _C // 4, bstep, 0)

            pltpu.sync_copy(ob, acc.at[dst_blk.at[jj]], add=True)
            @pl.when((i + 1) % _BLK != 0)
            def _():
                pltpu.async_copy(k_hbm.at[dst_blk.at[jj + 1]], kb, sem_kv)
            return 0
        lax.fori_loop(0, chunks, chunk, 0)

        plsc.subcore_barrier()
        for jj in range(rr_iters):
            c = jj * _NS + sid
            @pl.when(c < row_chunks)
            def _():
                sl = pl.ds(c * _C, _C)
                pltpu.sync_copy(acc.at[sl], out_hbm.at[cid, sl])

    return sc_edge


def kernel(feat, edge_index, bn_gamma, bn_beta, Wq, bq, Wk, Wv, We):
    N, D = feat.shape
    E = edge_index.shape[1]
    q, k, v = pl.pallas_call(
        _prep_body,
        out_shape=(
            jax.ShapeDtypeStruct((N, D), jnp.float32),
            jax.ShapeDtypeStruct((N, D), jnp.float32),
            jax.ShapeDtypeStruct((N, D), jnp.float32),
        ),
    )(feat, bn_gamma.reshape(1, -1), bn_beta.reshape(1, -1),
      Wq, bq.reshape(1, -1), Wk, Wv)

    partials = _make_sc_edge(N, E, D)(q, k, v,
                                      edge_index.reshape(-1, _C), We)

    rst = pl.pallas_call(
        _final_body,
        out_shape=jax.ShapeDtypeStruct((N, D), jnp.float32),
    )(partials)
    return rst


# speedup vs baseline: 1.3718x; 1.0262x over previous
import functools

import jax
import jax.numpy as jnp
from jax import lax
from jax.experimental import pallas as pl
from jax.experimental.pallas import tpu as pltpu
from jax.experimental.pallas import tpu_sc as plsc

_L = 16
_NC = 2
_NS = 16
_C = 80
_AW = 144


def _prep_body(feat_ref, g_ref, b_ref, wq_ref, bq_ref, wk_ref, wv_ref,
               q_ref, k_ref, v_ref):
    f = feat_ref[...]
    mu = jnp.mean(f, axis=0, keepdims=True)
    var = jnp.mean((f - mu) ** 2, axis=0, keepdims=True)
    x = (f - mu) * lax.rsqrt(var + 1e-5) * g_ref[...] + b_ref[...]
    dn = (((1,), (1,)), ((), ()))
    q_ref[...] = lax.dot_general(x, wq_ref[...], dn,
                                 precision=lax.Precision.HIGHEST) + bq_ref[...]
    k_ref[...] = lax.dot_general(x, wk_ref[...], dn,
                                 precision=lax.Precision.HIGHEST)
    v_ref[...] = lax.dot_general(x, wv_ref[...], dn,
                                 precision=lax.Precision.HIGHEST)


def _final_body(p_ref, o_ref):
    t = p_ref[0] + p_ref[1]
    s = t[:, 128:129]
    o_ref[...] = t[:, :128] / jnp.where(s > 0.0, s, 1.0)


_BLK = 25


def _make_sc_edge(N, E, D):
    per_tile = E // (_NC * _NS)
    chunks = per_tile // _C
    row_chunks = N // _C
    rr_iters = (row_chunks + _NS - 1) // _NS
    erows_per_tile = per_tile // _C
    dst_row0 = E // _C

    mesh = plsc.VectorSubcoreMesh(core_axis_name="c", subcore_axis_name="s",
                                  num_cores=_NC, num_subcores=_NS)

    @functools.partial(
        pl.kernel,
        out_type=jax.ShapeDtypeStruct((_NC, N, _AW), jnp.float32),
        mesh=mesh,
        scratch_types=[
            pltpu.VMEM((_BLK, _C), jnp.int32),
            pltpu.VMEM((_BLK, _C), jnp.int32),
            pltpu.VMEM((_C, D), jnp.float32),
            pltpu.VMEM((_C, D), jnp.float32),
            pltpu.VMEM((_C, _AW), jnp.float32),
            pltpu.VMEM((D,), jnp.float32),
            pltpu.VMEM((_C, _L), jnp.float32),
            pltpu.VMEM_SHARED((N, _AW), jnp.float32),
            pltpu.SemaphoreType.DMA,
            pltpu.SemaphoreType.DMA,
            pltpu.SemaphoreType.DMA,
            pltpu.SemaphoreType.DMA,
        ],
        compiler_params=pltpu.CompilerParams(use_tc_tiling_on_sc=False,
                                             needs_layout_passes=False,
                                             disable_bounds_checks=True),
    )
    def sc_edge(q_hbm, k_hbm, v_hbm, edge_hbm, we_hbm, out_hbm,
                src_blk, dst_blk, qb, kb, ob, web, wbuf, acc,
                sem_i, sem_q, sem_kv, sem_s):
        cid = lax.axis_index("c")
        sid = lax.axis_index("s")
        lane = lax.iota(jnp.int32, _L)
        zero16 = jnp.zeros((_L,), jnp.float32)
        rows_l = [p * _L + lane for p in range(_C // _L)]

        def zrow(r, _):
            for cc in range(_AW // _L):
                ob[r, pl.ds(cc * _L, _L)] = zero16
            return 0
        lax.fori_loop(0, _C, zrow, 0)
        for jj in range(rr_iters):
            c = jj * _NS + sid
            @pl.when(c < row_chunks)
            def _():
                pltpu.sync_copy(ob, acc.at[pl.ds(c * _C, _C)])

        pltpu.sync_copy(we_hbm.at[0], web)
        plsc.subcore_barrier()
        wevs = [web[pl.ds(j * _L, _L)] for j in range(D // _L)]
        lane0 = lane == 0

        erow0 = (cid * _NS + sid) * erows_per_tile

        def load_block(b):
            c1 = pltpu.async_copy(
                edge_hbm.at[pl.ds(erow0 + b * _BLK, _BLK)], src_blk, sem_i)
            c2 = pltpu.async_copy(
                edge_hbm.at[pl.ds(dst_row0 + erow0 + b * _BLK, _BLK)],
                dst_blk, sem_i)
            c1.wait()
            c2.wait()

        load_block(0)
        pltpu.async_copy(q_hbm.at[src_blk.at[0]], qb, sem_q)
        pltpu.async_copy(k_hbm.at[dst_blk.at[0]], kb, sem_kv)
        pltpu.async_copy(ob, acc.at[dst_blk.at[0]], sem_s, add=True)

        def drain_scatter():
            pltpu.make_async_copy(ob, acc.at[dst_blk.at[0]], sem_s).wait()

        def chunk(i, _):
            b = i // _BLK
            jj = i - b * _BLK
            boundary = jnp.logical_and(jj == 0, i > 0)

            @pl.when(boundary)
            def _():
                drain_scatter()
                load_block(b)
                pltpu.async_copy(q_hbm.at[src_blk.at[0]], qb, sem_q)
                pltpu.async_copy(k_hbm.at[dst_blk.at[0]], kb, sem_kv)

            pltpu.make_async_copy(q_hbm.at[src_blk.at[0]], qb, sem_q).wait()
            pltpu.make_async_copy(k_hbm.at[dst_blk.at[0]], kb, sem_kv).wait()

            def astep(t, _):
                for u in range(2):
                    e = t * 2 + u
                    acc = zero16
                    for j in range(D // _L):
                        qv = qb[e, pl.ds(j * _L, _L)]
                        kv = kb[e, pl.ds(j * _L, _L)]
                        sg = 1.0 / (1.0 + jnp.exp(-(qv + kv)))
                        acc = acc + wevs[j] * sg
                    s = jnp.sum(acc)
                    wv = jnp.exp(jnp.broadcast_to(s, (_L,)))
                    wbuf[e, :] = wv
                return 0
            lax.fori_loop(0, _C // 2, astep, 0)

            cp_v = pltpu.async_copy(v_hbm.at[src_blk.at[jj]], kb, sem_kv)
            @pl.when((i + 1) % _BLK != 0)
            def _():
                pltpu.async_copy(q_hbm.at[src_blk.at[jj + 1]], qb, sem_q)
            cp_v.wait()
            @pl.when(jnp.logical_not(boundary))
            def _():
                drain_scatter()

            def bstep(t, _):
                for u in range(4):
                    e = t * 4 + u
                    wv = wbuf[e, :]
                    for j in range(D // _L):
                        vv = kb[e, pl.ds(j * _L, _L)]
                        ob[e, pl.ds(j * _L, _L)] = vv * wv
                    ob[e, pl.ds(D, _L)] = jnp.where(lane0, wv, 0.0)
                return 0
            lax.fori_loop(0, _C // 4, bstep, 0)

            pltpu.async_copy(ob, acc.at[dst_blk.at[jj]], sem_s, add=True)
            @pl.when((i + 1) % _BLK != 0)
            def _():
                pltpu.async_copy(k_hbm.at[dst_blk.at[jj + 1]], kb, sem_kv)
            return 0
        lax.fori_loop(0, chunks, chunk, 0)
        drain_scatter()

        plsc.subcore_barrier()
        for jj in range(rr_iters):
            c = jj * _NS + sid
            @pl.when(c < row_chunks)
            def _():
                sl = pl.ds(c * _C, _C)
                pltpu.sync_copy(acc.at[sl], out_hbm.at[cid, sl])

    return sc_edge


def kernel(feat, edge_index, bn_gamma, bn_beta, Wq, bq, Wk, Wv, We):
    N, D = feat.shape
    E = edge_index.shape[1]
    q, k, v = pl.pallas_call(
        _prep_body,
        out_shape=(
            jax.ShapeDtypeStruct((N, D), jnp.float32),
            jax.ShapeDtypeStruct((N, D), jnp.float32),
            jax.ShapeDtypeStruct((N, D), jnp.float32),
        ),
    )(feat, bn_gamma.reshape(1, -1), bn_beta.reshape(1, -1),
      Wq, bq.reshape(1, -1), Wk, Wv)

    partials = _make_sc_edge(N, E, D)(q, k, v,
                                      edge_index.reshape(-1, _C), We)

    rst = pl.pallas_call(
        _final_body,
        out_shape=jax.ShapeDtypeStruct((N, D), jnp.float32),
    )(partials)
    return rst


# speedup vs baseline: 1.3751x; 1.0024x over previous
import functools

import jax
import jax.numpy as jnp
from jax import lax
from jax.experimental import pallas as pl
from jax.experimental.pallas import tpu as pltpu
from jax.experimental.pallas import tpu_sc as plsc

_L = 16
_NC = 2
_NS = 16
_C = 80
_AW = 144


def _prep_body(feat_ref, g_ref, b_ref, wq_ref, bq_ref, wk_ref, wv_ref,
               q_ref, k_ref, v_ref):
    f = feat_ref[...]
    mu = jnp.mean(f, axis=0, keepdims=True)
    var = jnp.mean((f - mu) ** 2, axis=0, keepdims=True)
    x = (f - mu) * lax.rsqrt(var + 1e-5) * g_ref[...] + b_ref[...]
    dn = (((1,), (1,)), ((), ()))
    q_ref[...] = lax.dot_general(x, wq_ref[...], dn,
                                 precision=lax.Precision.HIGHEST) + bq_ref[...]
    k_ref[...] = lax.dot_general(x, wk_ref[...], dn,
                                 precision=lax.Precision.HIGHEST)
    v_ref[...] = lax.dot_general(x, wv_ref[...], dn,
                                 precision=lax.Precision.HIGHEST)


def _final_body(p_ref, o_ref):
    t = p_ref[0] + p_ref[1]
    s = t[:, 128:129]
    o_ref[...] = t[:, :128] / jnp.where(s > 0.0, s, 1.0)


_BLK = 25


def _make_sc_edge(N, E, D):
    per_tile = E // (_NC * _NS)
    chunks = per_tile // _C
    row_chunks = N // _C
    rr_iters = (row_chunks + _NS - 1) // _NS
    erows_per_tile = per_tile // _C
    dst_row0 = E // _C

    mesh = plsc.VectorSubcoreMesh(core_axis_name="c", subcore_axis_name="s",
                                  num_cores=_NC, num_subcores=_NS)

    @functools.partial(
        pl.kernel,
        out_type=jax.ShapeDtypeStruct((_NC, N, _AW), jnp.float32),
        mesh=mesh,
        scratch_types=[
            pltpu.VMEM((_BLK, _C), jnp.int32),
            pltpu.VMEM((_BLK, _C), jnp.int32),
            pltpu.VMEM((_C, D), jnp.float32),
            pltpu.VMEM((_C, D), jnp.float32),
            pltpu.VMEM((_C, _AW), jnp.float32),
            pltpu.VMEM((D,), jnp.float32),
            pltpu.VMEM((_C, _L), jnp.float32),
            pltpu.VMEM_SHARED((N, _AW), jnp.float32),
            pltpu.SemaphoreType.DMA,
            pltpu.SemaphoreType.DMA,
            pltpu.SemaphoreType.DMA,
            pltpu.SemaphoreType.DMA,
        ],
        compiler_params=pltpu.CompilerParams(use_tc_tiling_on_sc=False,
                                             needs_layout_passes=False,
                                             disable_bounds_checks=True),
    )
    def sc_edge(q_hbm, k_hbm, v_hbm, edge_hbm, we_hbm, out_hbm,
                src_blk, dst_blk, qb, kb, ob, web, wbuf, acc,
                sem_i, sem_q, sem_kv, sem_s):
        cid = lax.axis_index("c")
        sid = lax.axis_index("s")
        lane = lax.iota(jnp.int32, _L)
        zero16 = jnp.zeros((_L,), jnp.float32)
        rows_l = [p * _L + lane for p in range(_C // _L)]

        def zrow(r, _):
            for cc in range(_AW // _L):
                ob[r, pl.ds(cc * _L, _L)] = zero16
            return 0
        lax.fori_loop(0, _C, zrow, 0)
        for jj in range(rr_iters):
            c = jj * _NS + sid
            @pl.when(c < row_chunks)
            def _():
                pltpu.sync_copy(ob, acc.at[pl.ds(c * _C, _C)])

        pltpu.sync_copy(we_hbm.at[0], web)
        plsc.subcore_barrier()
        wevs = [web[pl.ds(j * _L, _L)] for j in range(D // _L)]
        lane0 = lane == 0

        erow0 = (cid * _NS + sid) * erows_per_tile

        def load_block(b):
            c1 = pltpu.async_copy(
                edge_hbm.at[pl.ds(erow0 + b * _BLK, _BLK)], src_blk, sem_i)
            c2 = pltpu.async_copy(
                edge_hbm.at[pl.ds(dst_row0 + erow0 + b * _BLK, _BLK)],
                dst_blk, sem_i)
            c1.wait()
            c2.wait()

        load_block(0)
        pltpu.async_copy(q_hbm.at[src_blk.at[0]], qb, sem_q)
        pltpu.async_copy(k_hbm.at[dst_blk.at[0]], kb, sem_kv)
        pltpu.async_copy(ob, acc.at[dst_blk.at[0]], sem_s, add=True)

        def drain_scatter():
            pltpu.make_async_copy(ob, acc.at[dst_blk.at[0]], sem_s).wait()

        def chunk(i, _):
            b = i // _BLK
            jj = i - b * _BLK
            boundary = jnp.logical_and(jj == 0, i > 0)

            @pl.when(boundary)
            def _():
                drain_scatter()
                load_block(b)
                pltpu.async_copy(q_hbm.at[src_blk.at[0]], qb, sem_q)
                pltpu.async_copy(k_hbm.at[dst_blk.at[0]], kb, sem_kv)

            pltpu.make_async_copy(q_hbm.at[src_blk.at[0]], qb, sem_q).wait()
            pltpu.make_async_copy(k_hbm.at[dst_blk.at[0]], kb, sem_kv).wait()

            def astep(t, _):
                for u in range(2):
                    e = t * 2 + u
                    acc0 = zero16
                    acc1 = zero16
                    for j in range(0, D // _L, 2):
                        qv = qb[e, pl.ds(j * _L, _L)]
                        kv = kb[e, pl.ds(j * _L, _L)]
                        sg = 1.0 / (1.0 + jnp.exp(-(qv + kv)))
                        acc0 = acc0 + wevs[j] * sg
                        qv = qb[e, pl.ds((j + 1) * _L, _L)]
                        kv = kb[e, pl.ds((j + 1) * _L, _L)]
                        sg = 1.0 / (1.0 + jnp.exp(-(qv + kv)))
                        acc1 = acc1 + wevs[j + 1] * sg
                    s = jnp.sum(acc0 + acc1)
                    wv = jnp.exp(jnp.broadcast_to(s, (_L,)))
                    wbuf[e, :] = wv
                return 0
            lax.fori_loop(0, _C // 2, astep, 0)

            cp_v = pltpu.async_copy(v_hbm.at[src_blk.at[jj]], kb, sem_kv)
            @pl.when((i + 1) % _BLK != 0)
            def _():
                pltpu.async_copy(q_hbm.at[src_blk.at[jj + 1]], qb, sem_q)
            cp_v.wait()
            @pl.when(jnp.logical_not(boundary))
            def _():
                drain_scatter()

            def bstep(t, _):
                for u in range(4):
                    e = t * 4 + u
                    wv = wbuf[e, :]
                    for j in range(D // _L):
                        vv = kb[e, pl.ds(j * _L, _L)]
                        ob[e, pl.ds(j * _L, _L)] = vv * wv
                    ob[e, pl.ds(D, _L)] = jnp.where(lane0, wv, 0.0)
                return 0
            lax.fori_loop(0, _C // 4, bstep, 0)

            pltpu.async_copy(ob, acc.at[dst_blk.at[jj]], sem_s, add=True)
            @pl.when((i + 1) % _BLK != 0)
            def _():
                pltpu.async_copy(k_hbm.at[dst_blk.at[jj + 1]], kb, sem_kv)
            return 0
        lax.fori_loop(0, chunks, chunk, 0)
        drain_scatter()

        plsc.subcore_barrier()
        for jj in range(rr_iters):
            c = jj * _NS + sid
            @pl.when(c < row_chunks)
            def _():
                sl = pl.ds(c * _C, _C)
                pltpu.sync_copy(acc.at[sl], out_hbm.at[cid, sl])

    return sc_edge


def kernel(feat, edge_index, bn_gamma, bn_beta, Wq, bq, Wk, Wv, We):
    N, D = feat.shape
    E = edge_index.shape[1]
    q, k, v = pl.pallas_call(
        _prep_body,
        out_shape=(
            jax.ShapeDtypeStruct((N, D), jnp.float32),
            jax.ShapeDtypeStruct((N, D), jnp.float32),
            jax.ShapeDtypeStruct((N, D), jnp.float32),
        ),
    )(feat, bn_gamma.reshape(1, -1), bn_beta.reshape(1, -1),
      Wq, bq.reshape(1, -1), Wk, Wv)

    partials = _make_sc_edge(N, E, D)(q, k, v,
                                      edge_index.reshape(-1, _C), We)

    rst = pl.pallas_call(
        _final_body,
        out_shape=jax.ShapeDtypeStruct((N, D), jnp.float32),
    )(partials)
    return rst


# speedup vs baseline: 2.8642x; 2.0829x over previous
import functools

import jax
import jax.numpy as jnp
from jax import lax
from jax.experimental import pallas as pl
from jax.experimental.pallas import tpu as pltpu
from jax.experimental.pallas import tpu_sc as plsc

_L = 16
_NC = 2
_NS = 16
_C = 80
_AW = 144


def _prep_body(feat_ref, g_ref, b_ref, wq_ref, bq_ref, wk_ref, wv_ref,
               q_ref, k_ref, v_ref):
    f = feat_ref[...]
    mu = jnp.mean(f, axis=0, keepdims=True)
    var = jnp.mean((f - mu) ** 2, axis=0, keepdims=True)
    x = (f - mu) * lax.rsqrt(var + 1e-5) * g_ref[...] + b_ref[...]
    dn = (((1,), (1,)), ((), ()))
    q_ref[...] = lax.dot_general(x, wq_ref[...], dn,
                                 precision=lax.Precision.HIGHEST) + bq_ref[...]
    k_ref[...] = lax.dot_general(x, wk_ref[...], dn,
                                 precision=lax.Precision.HIGHEST)
    v_ref[...] = lax.dot_general(x, wv_ref[...], dn,
                                 precision=lax.Precision.HIGHEST)


def _final_body(p_ref, o_ref):
    t = p_ref[0] + p_ref[1]
    s = t[:, 128:129]
    o_ref[...] = t[:, :128] / jnp.where(s > 0.0, s, 1.0)


_BLK = 25


def _make_sc_edge(N, E, D):
    per_tile = E // (_NC * _NS)
    chunks = per_tile // _C
    row_chunks = N // _C
    rr_iters = (row_chunks + _NS - 1) // _NS
    erows_per_tile = per_tile // _C
    dst_row0 = E // _C

    mesh = plsc.VectorSubcoreMesh(core_axis_name="c", subcore_axis_name="s",
                                  num_cores=_NC, num_subcores=_NS)

    @functools.partial(
        pl.kernel,
        out_type=jax.ShapeDtypeStruct((_NC, N, _AW), jnp.float32),
        mesh=mesh,
        scratch_types=[
            pltpu.VMEM((_BLK, _C), jnp.int32),
            pltpu.VMEM((_BLK, _C), jnp.int32),
            pltpu.VMEM((_C, D), jnp.float32),
            pltpu.VMEM((_C, D), jnp.float32),
            pltpu.VMEM((_C, _AW), jnp.float32),
            pltpu.VMEM((D,), jnp.float32),
            pltpu.VMEM((_C, _L), jnp.float32),
            pltpu.VMEM_SHARED((N, _AW), jnp.float32),
            pltpu.SemaphoreType.DMA,
            pltpu.SemaphoreType.DMA,
            pltpu.SemaphoreType.DMA,
            pltpu.SemaphoreType.DMA,
        ],
        compiler_params=pltpu.CompilerParams(use_tc_tiling_on_sc=False,
                                             needs_layout_passes=False,
                                             disable_bounds_checks=True),
    )
    def sc_edge(q_hbm, k_hbm, v_hbm, edge_hbm, we_hbm, out_hbm,
                src_blk, dst_blk, qb, kb, ob, web, wbuf, acc,
                sem_i, sem_q, sem_kv, sem_s):
        cid = lax.axis_index("c")
        sid = lax.axis_index("s")
        lane = lax.iota(jnp.int32, _L)
        zero16 = jnp.zeros((_L,), jnp.float32)
        rows_l = [p * _L + lane for p in range(_C // _L)]

        def zrow(r, _):
            for cc in range(_AW // _L):
                ob[r, pl.ds(cc * _L, _L)] = zero16
            return 0
        lax.fori_loop(0, _C, zrow, 0)
        for jj in range(rr_iters):
            c = jj * _NS + sid
            @pl.when(c < row_chunks)
            def _():
                pltpu.sync_copy(ob, acc.at[pl.ds(c * _C, _C)])

        pltpu.sync_copy(we_hbm.at[0], web)
        plsc.subcore_barrier()
        wevs = [web[pl.ds(j * _L, _L)] for j in range(D // _L)]
        lane0 = lane == 0

        erow0 = (cid * _NS + sid) * erows_per_tile

        def load_block(b):
            c1 = pltpu.async_copy(
                edge_hbm.at[pl.ds(erow0 + b * _BLK, _BLK)], src_blk, sem_i)
            c2 = pltpu.async_copy(
                edge_hbm.at[pl.ds(dst_row0 + erow0 + b * _BLK, _BLK)],
                dst_blk, sem_i)
            c1.wait()
            c2.wait()

        load_block(0)
        pltpu.async_copy(q_hbm.at[src_blk.at[0]], qb, sem_q)
        pltpu.async_copy(k_hbm.at[dst_blk.at[0]], kb, sem_kv)
        pltpu.async_copy(ob, acc.at[dst_blk.at[0]], sem_s, add=True)

        def drain_scatter():
            pltpu.make_async_copy(ob, acc.at[dst_blk.at[0]], sem_s).wait()

        def chunk(i, _):
            b = i // _BLK
            jj = i - b * _BLK
            boundary = jnp.logical_and(jj == 0, i > 0)

            @pl.when(boundary)
            def _():
                drain_scatter()
                load_block(b)
                pltpu.async_copy(q_hbm.at[src_blk.at[0]], qb, sem_q)
                pltpu.async_copy(k_hbm.at[dst_blk.at[0]], kb, sem_kv)

            pltpu.make_async_copy(q_hbm.at[src_blk.at[0]], qb, sem_q).wait()
            pltpu.make_async_copy(k_hbm.at[dst_blk.at[0]], kb, sem_kv).wait()

            @plsc.parallel_loop(0, _C, step=1, unroll=2)
            def _(e):
                acc0 = zero16
                acc1 = zero16
                for j in range(0, D // _L, 2):
                    qv = qb[e, pl.ds(j * _L, _L)]
                    kv = kb[e, pl.ds(j * _L, _L)]
                    sg = 1.0 / (1.0 + jnp.exp(-(qv + kv)))
                    acc0 = acc0 + wevs[j] * sg
                    qv = qb[e, pl.ds((j + 1) * _L, _L)]
                    kv = kb[e, pl.ds((j + 1) * _L, _L)]
                    sg = 1.0 / (1.0 + jnp.exp(-(qv + kv)))
                    acc1 = acc1 + wevs[j + 1] * sg
                s = jnp.sum(acc0 + acc1)
                wv = jnp.exp(jnp.broadcast_to(s, (_L,)))
                wbuf[e, :] = wv

            cp_v = pltpu.async_copy(v_hbm.at[src_blk.at[jj]], kb, sem_kv)
            @pl.when((i + 1) % _BLK != 0)
            def _():
                pltpu.async_copy(q_hbm.at[src_blk.at[jj + 1]], qb, sem_q)
            cp_v.wait()
            @pl.when(jnp.logical_not(boundary))
            def _():
                drain_scatter()

            @plsc.parallel_loop(0, _C, step=1, unroll=4)
            def _(e):
                wv = wbuf[e, :]
                for j in range(D // _L):
                    vv = kb[e, pl.ds(j * _L, _L)]
                    ob[e, pl.ds(j * _L, _L)] = vv * wv
                ob[e, pl.ds(D, _L)] = jnp.where(lane0, wv, 0.0)

            pltpu.async_copy(ob, acc.at[dst_blk.at[jj]], sem_s, add=True)
            @pl.when((i + 1) % _BLK != 0)
            def _():
                pltpu.async_copy(k_hbm.at[dst_blk.at[jj + 1]], kb, sem_kv)
            return 0
        lax.fori_loop(0, chunks, chunk, 0)
        drain_scatter()

        plsc.subcore_barrier()
        for jj in range(rr_iters):
            c = jj * _NS + sid
            @pl.when(c < row_chunks)
            def _():
                sl = pl.ds(c * _C, _C)
                pltpu.sync_copy(acc.at[sl], out_hbm.at[cid, sl])

    return sc_edge


def kernel(feat, edge_index, bn_gamma, bn_beta, Wq, bq, Wk, Wv, We):
    N, D = feat.shape
    E = edge_index.shape[1]
    q, k, v = pl.pallas_call(
        _prep_body,
        out_shape=(
            jax.ShapeDtypeStruct((N, D), jnp.float32),
            jax.ShapeDtypeStruct((N, D), jnp.float32),
            jax.ShapeDtypeStruct((N, D), jnp.float32),
        ),
    )(feat, bn_gamma.reshape(1, -1), bn_beta.reshape(1, -1),
      Wq, bq.reshape(1, -1), Wk, Wv)

    partials = _make_sc_edge(N, E, D)(q, k, v,
                                      edge_index.reshape(-1, _C), We)

    rst = pl.pallas_call(
        _final_body,
        out_shape=jax.ShapeDtypeStruct((N, D), jnp.float32),
    )(partials)
    return rst


# speedup vs baseline: 2.9128x; 1.0170x over previous
import functools

import jax
import jax.numpy as jnp
from jax import lax
from jax.experimental import pallas as pl
from jax.experimental.pallas import tpu as pltpu
from jax.experimental.pallas import tpu_sc as plsc

_L = 16
_NC = 2
_NS = 16
_C = 80
_AW = 144


def _prep_body(feat_ref, g_ref, b_ref, wq_ref, bq_ref, wk_ref, wv_ref,
               q_ref, k_ref, v_ref):
    f = feat_ref[...]
    mu = jnp.mean(f, axis=0, keepdims=True)
    var = jnp.mean((f - mu) ** 2, axis=0, keepdims=True)
    x = (f - mu) * lax.rsqrt(var + 1e-5) * g_ref[...] + b_ref[...]
    dn = (((1,), (1,)), ((), ()))
    q_ref[...] = lax.dot_general(x, wq_ref[...], dn,
                                 precision=lax.Precision.HIGHEST) + bq_ref[...]
    k_ref[...] = lax.dot_general(x, wk_ref[...], dn,
                                 precision=lax.Precision.HIGHEST)
    v_ref[...] = lax.dot_general(x, wv_ref[...], dn,
                                 precision=lax.Precision.HIGHEST)


def _final_body(p_ref, o_ref):
    t = p_ref[0] + p_ref[1]
    s = t[:, 128:129]
    o_ref[...] = t[:, :128] / jnp.where(s > 0.0, s, 1.0)


_BLK = 25


def _make_sc_edge(N, E, D):
    per_tile = E // (_NC * _NS)
    chunks = per_tile // _C
    row_chunks = N // _C
    rr_iters = (row_chunks + _NS - 1) // _NS
    erows_per_tile = per_tile // _C
    dst_row0 = E // _C

    mesh = plsc.VectorSubcoreMesh(core_axis_name="c", subcore_axis_name="s",
                                  num_cores=_NC, num_subcores=_NS)

    @functools.partial(
        pl.kernel,
        out_type=jax.ShapeDtypeStruct((_NC, N, _AW), jnp.float32),
        mesh=mesh,
        scratch_types=[
            pltpu.VMEM((_BLK, _C), jnp.int32),
            pltpu.VMEM((_BLK, _C), jnp.int32),
            pltpu.VMEM((_C, D), jnp.float32),
            pltpu.VMEM((_C, D), jnp.float32),
            pltpu.VMEM((_C, _AW), jnp.float32),
            pltpu.VMEM((D,), jnp.float32),
            pltpu.VMEM((_C, _L), jnp.float32),
            pltpu.VMEM_SHARED((N, _AW), jnp.float32),
            pltpu.SemaphoreType.DMA,
            pltpu.SemaphoreType.DMA,
            pltpu.SemaphoreType.DMA,
            pltpu.SemaphoreType.DMA,
        ],
        compiler_params=pltpu.CompilerParams(use_tc_tiling_on_sc=False,
                                             needs_layout_passes=False,
                                             disable_bounds_checks=True),
    )
    def sc_edge(q_hbm, k_hbm, v_hbm, edge_hbm, we_hbm, out_hbm,
                src_blk, dst_blk, qb, kb, ob, web, wbuf, acc,
                sem_i, sem_q, sem_kv, sem_s):
        cid = lax.axis_index("c")
        sid = lax.axis_index("s")
        lane = lax.iota(jnp.int32, _L)
        zero16 = jnp.zeros((_L,), jnp.float32)
        rows_l = [p * _L + lane for p in range(_C // _L)]

        def zrow(r, _):
            for cc in range(_AW // _L):
                ob[r, pl.ds(cc * _L, _L)] = zero16
            return 0
        lax.fori_loop(0, _C, zrow, 0)
        for jj in range(rr_iters):
            c = jj * _NS + sid
            @pl.when(c < row_chunks)
            def _():
                pltpu.sync_copy(ob, acc.at[pl.ds(c * _C, _C)])

        pltpu.sync_copy(we_hbm.at[0], web)
        plsc.subcore_barrier()
        wevs = [web[pl.ds(j * _L, _L)] for j in range(D // _L)]
        lane0 = lane == 0

        erow0 = (cid * _NS + sid) * erows_per_tile

        def load_block(b):
            c1 = pltpu.async_copy(
                edge_hbm.at[pl.ds(erow0 + b * _BLK, _BLK)], src_blk, sem_i)
            c2 = pltpu.async_copy(
                edge_hbm.at[pl.ds(dst_row0 + erow0 + b * _BLK, _BLK)],
                dst_blk, sem_i)
            c1.wait()
            c2.wait()

        load_block(0)
        pltpu.async_copy(q_hbm.at[src_blk.at[0]], qb, sem_q)
        pltpu.async_copy(k_hbm.at[dst_blk.at[0]], kb, sem_kv)
        pltpu.async_copy(ob, acc.at[dst_blk.at[0]], sem_s, add=True)

        def drain_scatter():
            pltpu.make_async_copy(ob, acc.at[dst_blk.at[0]], sem_s).wait()

        def chunk(i, _):
            b = i // _BLK
            jj = i - b * _BLK
            boundary = jnp.logical_and(jj == 0, i > 0)

            @pl.when(boundary)
            def _():
                drain_scatter()
                load_block(b)
                pltpu.async_copy(q_hbm.at[src_blk.at[0]], qb, sem_q)
                pltpu.async_copy(k_hbm.at[dst_blk.at[0]], kb, sem_kv)

            pltpu.make_async_copy(q_hbm.at[src_blk.at[0]], qb, sem_q).wait()
            pltpu.make_async_copy(k_hbm.at[dst_blk.at[0]], kb, sem_kv).wait()

            @plsc.parallel_loop(0, _C, step=1, unroll=4)
            def _(e):
                acc0 = zero16
                acc1 = zero16
                for j in range(0, D // _L, 2):
                    qv = qb[e, pl.ds(j * _L, _L)]
                    kv = kb[e, pl.ds(j * _L, _L)]
                    sg = 1.0 / (1.0 + jnp.exp(-(qv + kv)))
                    acc0 = acc0 + wevs[j] * sg
                    qv = qb[e, pl.ds((j + 1) * _L, _L)]
                    kv = kb[e, pl.ds((j + 1) * _L, _L)]
                    sg = 1.0 / (1.0 + jnp.exp(-(qv + kv)))
                    acc1 = acc1 + wevs[j + 1] * sg
                s = jnp.sum(acc0 + acc1)
                wv = jnp.exp(jnp.broadcast_to(s, (_L,)))
                wbuf[e, :] = wv

            cp_v = pltpu.async_copy(v_hbm.at[src_blk.at[jj]], kb, sem_kv)
            @pl.when((i + 1) % _BLK != 0)
            def _():
                pltpu.async_copy(q_hbm.at[src_blk.at[jj + 1]], qb, sem_q)
            cp_v.wait()
            @pl.when(jnp.logical_not(boundary))
            def _():
                drain_scatter()

            @plsc.parallel_loop(0, _C, step=1, unroll=4)
            def _(e):
                wv = wbuf[e, :]
                for j in range(D // _L):
                    vv = kb[e, pl.ds(j * _L, _L)]
                    ob[e, pl.ds(j * _L, _L)] = vv * wv
                ob[e, pl.ds(D, _L)] = jnp.where(lane0, wv, 0.0)

            pltpu.async_copy(ob, acc.at[dst_blk.at[jj]], sem_s, add=True)
            @pl.when((i + 1) % _BLK != 0)
            def _():
                pltpu.async_copy(k_hbm.at[dst_blk.at[jj + 1]], kb, sem_kv)
            return 0
        lax.fori_loop(0, chunks, chunk, 0)
        drain_scatter()

        plsc.subcore_barrier()
        for jj in range(rr_iters):
            c = jj * _NS + sid
            @pl.when(c < row_chunks)
            def _():
                sl = pl.ds(c * _C, _C)
                pltpu.sync_copy(acc.at[sl], out_hbm.at[cid, sl])

    return sc_edge


def kernel(feat, edge_index, bn_gamma, bn_beta, Wq, bq, Wk, Wv, We):
    N, D = feat.shape
    E = edge_index.shape[1]
    q, k, v = pl.pallas_call(
        _prep_body,
        out_shape=(
            jax.ShapeDtypeStruct((N, D), jnp.float32),
            jax.ShapeDtypeStruct((N, D), jnp.float32),
            jax.ShapeDtypeStruct((N, D), jnp.float32),
        ),
    )(feat, bn_gamma.reshape(1, -1), bn_beta.reshape(1, -1),
      Wq, bq.reshape(1, -1), Wk, Wv)

    partials = _make_sc_edge(N, E, D)(q, k, v,
                                      edge_index.reshape(-1, _C), We)

    rst = pl.pallas_call(
        _final_body,
        out_shape=jax.ShapeDtypeStruct((N, D), jnp.float32),
    )(partials)
    return rst
